# Initial kernel scaffold; baseline (speedup 1.0000x reference)
import jax
import jax.numpy as jnp
from jax.experimental import pallas as pl


def kernel(edges_by_rating, deg_user, deg_item, X_u, X_v, W_i2u, W_u2i, Wou_w, Wou_b, Woi_w, Woi_b):
    raise NotImplementedError("write your pallas kernel here")


# speedup vs baseline: 2.1762x; 2.1762x over previous
import functools

import jax
import jax.numpy as jnp
from jax import lax
from jax.experimental import pallas as pl
from jax.experimental.pallas import tpu as pltpu
from jax.experimental.pallas import tpu_sc as plsc

N = 100000
RR = 5
E = 125000
D_IN = 128
D_OUT = 64
NS = 16
TPT = 8192
EP = NS * TPT
KB = 2048
CH = 25000
ACC_ROWS = 25088
TRASH = 25024
G = 128
ZR = ACC_ROWS // NS
PAD_IDX = 1 << 30
_HI = jax.lax.Precision.HIGHEST

MB = 1000


def _wcomb_body(wiu_ref, wui_ref, wou_ref, woi_ref, a_ref, b_ref):
    for r in range(RR):
        a_ref[r] = lax.dot(wiu_ref[r], wou_ref[...], precision=_HI)
        b_ref[r] = lax.dot(wui_ref[r], woi_ref[...], precision=_HI)


def _wcomb(W_i2u, W_u2i, Wou_w, Woi_w):
    return pl.pallas_call(
        _wcomb_body,
        out_shape=[jax.ShapeDtypeStruct((RR, D_IN, D_OUT), jnp.float32),
                   jax.ShapeDtypeStruct((RR, D_IN, D_OUT), jnp.float32)],
    )(W_i2u, W_u2i, Wou_w, Woi_w)


def _mm_body(x_ref, deg_ref, w_ref, o_ref):
    s = lax.rsqrt(jnp.clip(deg_ref[...], 1e-10, None))
    xb = x_ref[...]
    for r in range(RR):
        o_ref[r] = lax.dot(xb, w_ref[r], precision=_HI) * s


def _mm(X, deg2d, W):
    return pl.pallas_call(
        _mm_body,
        grid=(N // MB,),
        in_specs=[
            pl.BlockSpec((MB, D_IN), lambda i: (i, 0)),
            pl.BlockSpec((MB, 1), lambda i: (i, 0)),
            pl.BlockSpec((RR, D_IN, D_OUT), lambda i: (0, 0, 0)),
        ],
        out_specs=pl.BlockSpec((RR, MB, D_OUT), lambda i: (0, i, 0)),
        out_shape=jax.ShapeDtypeStruct((RR, N, D_OUT), jnp.float32),
    )(X, deg2d, W)


def _fin_body(acc_ref, deg_ref, b_ref, o_ref):
    s = lax.rsqrt(jnp.clip(deg_ref[...], 1e-10, None))
    o_ref[...] = jnp.maximum(acc_ref[...] * s + b_ref[...], 0.0)


def _fin(acc, deg2d, bias2d):
    return pl.pallas_call(
        _fin_body,
        grid=(N // MB,),
        in_specs=[
            pl.BlockSpec((MB, D_OUT), lambda i: (i, 0)),
            pl.BlockSpec((MB, 1), lambda i: (i, 0)),
            pl.BlockSpec((1, D_OUT), lambda i: (0, 0)),
        ],
        out_specs=pl.BlockSpec((MB, D_OUT), lambda i: (i, 0)),
        out_shape=jax.ShapeDtypeStruct((N, D_OUT), jnp.float32),
    )(acc, deg2d, bias2d)


@functools.partial(
    pl.kernel,
    mesh=plsc.VectorSubcoreMesh(core_axis_name="c", subcore_axis_name="s"),
    out_type=[jax.ShapeDtypeStruct((N, D_OUT), jnp.float32),
              jax.ShapeDtypeStruct((N, D_OUT), jnp.float32)],
    compiler_params=pltpu.CompilerParams(needs_layout_passes=False,
                                         use_tc_tiling_on_sc=False),
    scratch_types=[
        pltpu.VMEM_SHARED((ACC_ROWS, D_OUT), jnp.float32),
        pltpu.VMEM((KB,), jnp.int32),
        pltpu.VMEM((KB,), jnp.int32),
        pltpu.VMEM((KB + G,), jnp.int32),
        pltpu.VMEM((KB + G,), jnp.int32),
        pltpu.VMEM((G, D_OUT), jnp.float32),
        pltpu.VMEM((16,), jnp.int32),
        pltpu.SemaphoreType.DMA,
    ],
)
def _sc_scatter(edges, yv, yu, zsrc, accu_out, accv_out,
                acc, ublk, vblk, sdst, ssrc, rowbuf, cbuf, sem):
    cid = lax.axis_index("c")
    sid = lax.axis_index("s")

    trash16 = jnp.full((16,), TRASH, jnp.int32)
    zero16 = jnp.zeros((16,), jnp.int32)

    for direction in range(2):
        y = yv if direction == 0 else yu
        out = accu_out if direction == 0 else accv_out
        d_dst = direction
        d_src = 1 - direction
        for p in range(2):
            chunk = cid * 2 + p
            lo = chunk * CH
            pltpu.sync_copy(zsrc, acc.at[pl.ds(sid * ZR, ZR)])
            plsc.subcore_barrier()

            def rating_body(r, _):
                def block_body(b, _):
                    off = sid * TPT + b * KB
                    pltpu.sync_copy(edges.at[r * 2 + d_dst, 0, pl.ds(off, KB)], ublk)
                    pltpu.sync_copy(edges.at[r * 2 + d_src, 0, pl.ds(off, KB)], vblk)
                    srcbase = r * N

                    def grp(i, nv):
                        u16 = ublk[pl.ds(i * 16, 16)]
                        s16 = vblk[pl.ds(i * 16, 16)]
                        m = (u16 >= lo) & (u16 < lo + CH)
                        mi = m.astype(jnp.int32)
                        cs = plsc.cumsum(mi)
                        idx = nv + cs - mi
                        plsc.store_scatter(sdst, [idx], u16 - lo, mask=m)
                        plsc.store_scatter(ssrc, [idx], s16 + srcbase, mask=m)
                        return nv + plsc.cummax(lax.rev(cs, (0,)))

                    nv = lax.fori_loop(0, KB // 16, grp, jnp.zeros((16,), jnp.int32))
                    cbuf[...] = nv
                    n = cbuf[...][0]

                    for k8 in range(G // 16):
                        sdst[pl.ds(n + k8 * 16, 16)] = trash16
                        ssrc[pl.ds(n + k8 * 16, 16)] = zero16
                    ngrp = (n + G - 1) // G

                    def stream_grp(g, _):
                        pltpu.async_copy(y.at[ssrc.at[pl.ds(g * G, G)]],
                                         rowbuf, sem).wait()
                        pltpu.sync_copy(rowbuf, acc.at[sdst.at[pl.ds(g * G, G)]],
                                        add=True)
                        return 0

                    lax.fori_loop(0, ngrp, stream_grp, 0)
                    return 0

                return lax.fori_loop(0, TPT // KB, block_body, 0)

            lax.fori_loop(0, RR, rating_body, 0)
            plsc.subcore_barrier()

            wbase = sid * 1560
            for (o, sz) in tuple((k * G, G) for k in range(12)) + ((1536, 24),):
                pltpu.sync_copy(acc.at[pl.ds(wbase + o, sz)],
                                rowbuf.at[pl.ds(0, sz)])
                pltpu.sync_copy(rowbuf.at[pl.ds(0, sz)],
                                out.at[pl.ds(lo + wbase + o, sz)])

            @pl.when(sid < 5)
            def _():
                rem = 16 * 1560 + sid * 8
                pltpu.sync_copy(acc.at[pl.ds(rem, 8)], rowbuf.at[pl.ds(0, 8)])
                pltpu.sync_copy(rowbuf.at[pl.ds(0, 8)], out.at[pl.ds(lo + rem, 8)])

            plsc.subcore_barrier()


def kernel(edges_by_rating, deg_user, deg_item, X_u, X_v,
           W_i2u, W_u2i, Wou_w, Wou_b, Woi_w, Woi_b):
    A, B = _wcomb(W_i2u, W_u2i, Wou_w, Woi_w)
    du = deg_user.reshape(N, 1)
    dv = deg_item.reshape(N, 1)
    Yv = _mm(X_v, dv, A).reshape(RR * N, D_OUT)
    Yu = _mm(X_u, du, B).reshape(RR * N, D_OUT)
    edges_p = jnp.pad(edges_by_rating, ((0, 0), (0, 0), (0, EP - E)),
                      constant_values=PAD_IDX).reshape(RR * 2, 1, EP)
    zsrc = jnp.zeros((ZR, D_OUT), jnp.float32)
    acc_u, acc_v = _sc_scatter(edges_p, Yv, Yu, zsrc)
    h_u = _fin(acc_u, du, Wou_b.reshape(1, D_OUT))
    h_v = _fin(acc_v, dv, Woi_b.reshape(1, D_OUT))
    return (h_u, h_v)
